# baseline (speedup 1.0000x reference)
import functools

import jax
import jax.numpy as jnp
from jax import lax
from jax.experimental import pallas as pl
from jax.experimental.pallas import tpu as pltpu
from jax.experimental.pallas import tpu_sc as plsc

_LANES = 16
_NW = 32
_CS = 8
_NBUF = 3


@functools.lru_cache(maxsize=None)
def _build(B, S, D):
    assert S % _NW == 0
    s_per_w = S // _NW
    assert s_per_w % _CS == 0
    n_chunks = s_per_w // _CS
    C = B * _CS
    mesh = plsc.VectorSubcoreMesh(core_axis_name="c", subcore_axis_name="s")

    @functools.partial(
        pl.kernel,
        out_type=jax.ShapeDtypeStruct((B * S, D), jnp.float32),
        mesh=mesh,
        scratch_types=[
            pltpu.VMEM((B * s_per_w,), jnp.int32),
            pltpu.VMEM((_NBUF * C, D), jnp.float32),
            pltpu.VMEM((_NBUF * _CS, D), jnp.float32),
        ]
        + [pltpu.SemaphoreType.DMA] * (3 * _NBUF),
    )
    def emb(ids_hbm, wte_hbm, wpe_hbm, out_hbm, idx_all, rows_v, pos_v, *sems):
        gsem = sems[0:_NBUF]
        psem = sems[_NBUF:2 * _NBUF]
        wsem = sems[2 * _NBUF:3 * _NBUF]
        wid = lax.axis_index("s") * 2 + lax.axis_index("c")
        s0 = wid * s_per_w

        pltpu.sync_copy(ids_hbm.at[pl.ds(wid * (B * s_per_w), B * s_per_w)],
                        idx_all)

        def start(g, p):
            gd = pltpu.async_copy(
                wte_hbm.at[idx_all.at[pl.ds(g * C, C)]],
                rows_v.at[pl.ds(p * C, C)], gsem[p])
            pd = pltpu.async_copy(
                wpe_hbm.at[pl.ds(s0 + g * _CS, _CS)],
                pos_v.at[pl.ds(p * _CS, _CS)], psem[p])
            return gd, pd

        def write(g, p):
            return [
                pltpu.async_copy(
                    rows_v.at[pl.ds(p * C + b * _CS, _CS)],
                    out_hbm.at[pl.ds(b * S + s0 + g * _CS, _CS)], wsem[p])
                for b in range(B)
            ]

        gdesc = {}
        wdesc = {}
        for g in range(min(2, n_chunks)):
            gdesc[g] = start(g, g % _NBUF)

        for g in range(n_chunks):
            p = g % _NBUF
            gd, pd = gdesc.pop(g)
            gd.wait()
            pd.wait()

            @pl.loop(0, _CS * (D // _LANES), unroll=2)
            def _add(i):
                r = i // (D // _LANES)
                sl = pl.ds((i % (D // _LANES)) * _LANES, _LANES)
                v = pos_v[p * _CS + r, sl]
                for b in range(B):
                    rows_v[p * C + b * _CS + r, sl] = (
                        rows_v[p * C + b * _CS + r, sl] + v)

            wdesc[g] = write(g, p)
            if g + 2 < n_chunks:
                if g - 1 >= 0:
                    for d in wdesc.pop(g - 1):
                        d.wait()
                gdesc[g + 2] = start(g + 2, (g + 2) % _NBUF)

        for g in sorted(wdesc):
            for d in wdesc.pop(g):
                d.wait()

    return emb


def kernel(input_ids, wte, wpe):
    B, S = input_ids.shape
    D = wte.shape[1]
    s_per_w = S // _NW
    ids = (input_ids.astype(jnp.int32)
           .reshape(B, _NW, s_per_w // _CS, _CS)
           .transpose(1, 2, 0, 3)
           .reshape(-1))
    out = _build(B, S, D)(ids, wte, wpe)
    return out.reshape(B, S, D)


# speedup vs baseline: 1.4157x; 1.4157x over previous
import functools

import jax
import jax.numpy as jnp
from jax import lax
from jax.experimental import pallas as pl
from jax.experimental.pallas import tpu as pltpu
from jax.experimental.pallas import tpu_sc as plsc

_LANES = 16
_NW = 32
_CS = 4
_NBUF = 4


@functools.lru_cache(maxsize=None)
def _build(B, S, D):
    assert S % _NW == 0
    s_per_w = S // _NW
    assert s_per_w % (_NBUF * _CS) == 0
    n_chunks = s_per_w // _CS
    C = B * _CS
    DL = D // _LANES
    mesh = plsc.VectorSubcoreMesh(core_axis_name="c", subcore_axis_name="s")

    @functools.partial(
        pl.kernel,
        out_type=jax.ShapeDtypeStruct((B * S, D), jnp.float32),
        mesh=mesh,
        scratch_types=[
            pltpu.VMEM((B * s_per_w,), jnp.int32),
            pltpu.VMEM((_NBUF * C, D), jnp.float32),
            pltpu.VMEM((_NBUF * _CS, D), jnp.float32),
        ]
        + [pltpu.SemaphoreType.DMA] * (3 * _NBUF),
    )
    def emb(ids_hbm, wte_hbm, wpe_hbm, out_hbm, idx_all, rows_v, pos_v, *sems):
        gsem = sems[0:_NBUF]
        psem = sems[_NBUF:2 * _NBUF]
        wsem = sems[2 * _NBUF:3 * _NBUF]
        wid = lax.axis_index("s") * 2 + lax.axis_index("c")
        s0 = wid * s_per_w

        pltpu.sync_copy(ids_hbm.at[pl.ds(wid * (B * s_per_w), B * s_per_w)],
                        idx_all)

        def start(g, p):
            pltpu.async_copy(wte_hbm.at[idx_all.at[pl.ds(g * C, C)]],
                             rows_v.at[pl.ds(p * C, C)], gsem[p])
            pltpu.async_copy(wpe_hbm.at[pl.ds(s0 + g * _CS, _CS)],
                             pos_v.at[pl.ds(p * _CS, _CS)], psem[p])

        def wait_in(p):
            pltpu.make_async_copy(wte_hbm.at[pl.ds(0, C)],
                                  rows_v.at[pl.ds(p * C, C)], gsem[p]).wait()
            pltpu.make_async_copy(wpe_hbm.at[pl.ds(0, _CS)],
                                  pos_v.at[pl.ds(p * _CS, _CS)],
                                  psem[p]).wait()

        def issue_writes(g, p):
            for b in range(B):
                pltpu.async_copy(rows_v.at[pl.ds(p * C + b * _CS, _CS)],
                                 out_hbm.at[pl.ds(b * S + s0 + g * _CS, _CS)],
                                 wsem[p])

        def wait_writes(p):
            pltpu.make_async_copy(rows_v.at[pl.ds(p * C, C)],
                                  out_hbm.at[pl.ds(0, C)], wsem[p]).wait()

        def add(p):
            @pl.loop(0, _CS * DL, unroll=8)
            def _add(i):
                r = i // DL
                sl = pl.ds((i % DL) * _LANES, _LANES)
                v = pos_v[p * _CS + r, sl]
                for b in range(B):
                    plsc.addupdate(rows_v.at[p * C + b * _CS + r, sl], v)

        start(0, 0)
        start(1, 1)
        for g in (0, 1):
            wait_in(g)
            add(g)
            issue_writes(g, g)
            start(g + 2, g + 2)

        @pl.loop(0, (n_chunks - 4) // _NBUF)
        def _group(h):
            g0 = h * _NBUF + 2
            for ph in range(_NBUF):
                p = (ph + 2) % _NBUF
                q = ph
                g = g0 + ph
                wait_in(p)
                add(p)
                issue_writes(g, p)
                wait_writes(q)
                start(g + 2, q)

        for k in (2, 1):
            g = n_chunks - k
            p = g % _NBUF
            wait_in(p)
            add(p)
            issue_writes(g, p)
        for p in range(_NBUF):
            wait_writes(p)

    return emb


def kernel(input_ids, wte, wpe):
    B, S = input_ids.shape
    D = wte.shape[1]
    s_per_w = S // _NW
    ids = (input_ids.astype(jnp.int32)
           .reshape(B, _NW, s_per_w // _CS, _CS)
           .transpose(1, 2, 0, 3)
           .reshape(-1))
    out = _build(B, S, D)(ids, wte, wpe)
    return out.reshape(B, S, D)


# speedup vs baseline: 1.5176x; 1.0720x over previous
import functools

import jax
import jax.numpy as jnp
from jax import lax
from jax.experimental import pallas as pl
from jax.experimental.pallas import tpu as pltpu
from jax.experimental.pallas import tpu_sc as plsc

_LANES = 16
_NW = 32
_CS = 8
_NBUF = 3


@functools.lru_cache(maxsize=None)
def _build(B, S, D):
    assert S % _NW == 0
    s_per_w = S // _NW
    assert s_per_w % _CS == 0
    n_chunks = s_per_w // _CS
    C = B * _CS
    mesh = plsc.VectorSubcoreMesh(core_axis_name="c", subcore_axis_name="s")

    @functools.partial(
        pl.kernel,
        out_type=jax.ShapeDtypeStruct((B, S, D), jnp.float32),
        mesh=mesh,
        scratch_types=[
            pltpu.VMEM((B * s_per_w,), jnp.int32),
            pltpu.VMEM((_NBUF, B, _CS, D), jnp.float32),
            pltpu.VMEM((_NBUF * _CS, D), jnp.float32),
        ]
        + [pltpu.SemaphoreType.DMA] * (3 * _NBUF),
    )
    def emb(ids_hbm, wte_hbm, wpe_hbm, out_hbm, idx_all, rows_v, pos_v, *sems):
        gsem = sems[0:_NBUF]
        psem = sems[_NBUF:2 * _NBUF]
        wsem = sems[2 * _NBUF:3 * _NBUF]
        wid = lax.axis_index("s") * 2 + lax.axis_index("c")
        s0 = wid * s_per_w

        pltpu.sync_copy(ids_hbm.at[pl.ds(wid * (B * s_per_w), B * s_per_w)],
                        idx_all)

        def start(g, p):
            gd = [
                pltpu.async_copy(
                    wte_hbm.at[idx_all.at[pl.ds(g * C + b * _CS, _CS)]],
                    rows_v.at[p, b], gsem[p])
                for b in range(B)
            ]
            pd = pltpu.async_copy(
                wpe_hbm.at[pl.ds(s0 + g * _CS, _CS)],
                pos_v.at[pl.ds(p * _CS, _CS)], psem[p])
            return gd, pd

        def write(g, p):
            return [pltpu.async_copy(
                rows_v.at[p],
                out_hbm.at[:, pl.ds(s0 + g * _CS, _CS), :], wsem[p])]

        gdesc = {}
        wdesc = {}
        for g in range(min(2, n_chunks)):
            gdesc[g] = start(g, g % _NBUF)

        for g in range(n_chunks):
            p = g % _NBUF
            gd, pd = gdesc.pop(g)
            for d in gd:
                d.wait()
            pd.wait()

            @pl.loop(0, _CS * (D // _LANES), unroll=8)
            def _add(i):
                r = i // (D // _LANES)
                sl = pl.ds((i % (D // _LANES)) * _LANES, _LANES)
                v = pos_v[p * _CS + r, sl]
                for b in range(B):
                    plsc.addupdate(rows_v.at[p, b, r, sl], v)

            wdesc[g] = write(g, p)
            if g + 2 < n_chunks:
                if g - 1 >= 0:
                    for d in wdesc.pop(g - 1):
                        d.wait()
                gdesc[g + 2] = start(g + 2, (g + 2) % _NBUF)

        for g in sorted(wdesc):
            for d in wdesc.pop(g):
                d.wait()

    return emb


def kernel(input_ids, wte, wpe):
    B, S = input_ids.shape
    D = wte.shape[1]
    s_per_w = S // _NW
    ids = (input_ids.astype(jnp.int32)
           .reshape(B, _NW, s_per_w // _CS, _CS)
           .transpose(1, 2, 0, 3)
           .reshape(-1))
    out = _build(B, S, D)(ids, wte, wpe)
    return out.reshape(B, S, D)
